# baseline (speedup 1.0000x reference)
import jax
import jax.numpy as jnp
from jax.experimental import pallas as pl
from jax.experimental.pallas import tpu as pltpu

TM = 128
CH = 128


def _proj_kernel(xb_ref, w_ref, dinv_ref, o_ref):
    acc = jnp.dot(xb_ref[...], w_ref[...], preferred_element_type=jnp.float32)
    o_ref[...] = acc * dinv_ref[:, :1]


def _spmm_accumulate(i, ts_ref, src_ref, dst_ref, m3_ref, mblk_ref, acc_ref,
                     g_ref):
    acc_ref[...] = mblk_ref[...].astype(jnp.float32)
    start = ts_ref[i]
    end = ts_ref[i + 1]
    c0 = start // CH
    c1 = (end + CH - 1) // CH
    row_iota = jax.lax.broadcasted_iota(jnp.int32, (TM, 2 * CH), 0)
    lane_iota = jax.lax.broadcasted_iota(jnp.int32, (1, 2 * CH), 1)
    base = i * TM

    def body(p, carry):
        ec = p * (2 * CH)
        for j in range(CH):
            w = src_ref[p * CH + j]
            g_ref[2 * j, :] = m3_ref[w & 0xFFFF, 0, :].astype(jnp.bfloat16)
            g_ref[2 * j + 1, :] = m3_ref[w >> 16, 0, :].astype(jnp.bfloat16)
        ev = ec + lane_iota
        dl = jnp.concatenate(
            [dst_ref[2 * p, 0, :][None, :],
             dst_ref[2 * p + 1, 0, :][None, :]], axis=1) - base
        valid = (ev >= start) & (ev < end)
        u = jnp.where(valid & (row_iota == dl), 1.0, 0.0).astype(jnp.bfloat16)
        acc_ref[...] += jnp.dot(u, g_ref[...],
                                preferred_element_type=jnp.float32)
        return carry

    jax.lax.fori_loop(c0 // 2, (c1 + 1) // 2, body, 0)
    return acc_ref[...]


def _layer_kernel(ts_ref, src_ref, dst_ref, m3_ref, mblk_ref, dinv_ref, b_ref,
                  wn_ref, o_ref, acc_ref, g_ref):
    i = pl.program_id(0) * pl.num_programs(1) + pl.program_id(1)
    acc = _spmm_accumulate(i, ts_ref, src_ref, dst_ref, m3_ref, mblk_ref,
                           acc_ref, g_ref)
    dinv = dinv_ref[:, :1]
    h = jnp.maximum(acc * dinv + b_ref[...], 0.0)
    g1 = jnp.dot(h.astype(jnp.bfloat16), wn_ref[...],
                 preferred_element_type=jnp.float32)
    o_ref[...] = g1 * dinv


def _final_kernel(ts_ref, src_ref, dst_ref, m3_ref, mblk_ref, dinv_ref, b_ref,
                  wfc_ref, bfc_ref, z_ref, p_ref, acc_ref, g_ref):
    i = pl.program_id(0) * pl.num_programs(1) + pl.program_id(1)
    acc = _spmm_accumulate(i, ts_ref, src_ref, dst_ref, m3_ref, mblk_ref,
                           acc_ref, g_ref)
    dinv = dinv_ref[:, :1]
    z = acc * dinv + b_ref[...]
    z_ref[...] = z
    y = jnp.dot(jnp.maximum(z, 0.0).astype(jnp.bfloat16), wfc_ref[...],
                preferred_element_type=jnp.float32) + bfc_ref[...]
    m = jnp.max(y, axis=1, keepdims=True)
    e = jnp.exp(y - m)
    p_ref[...] = e / jnp.sum(e, axis=1, keepdims=True)


def kernel(x, edge_index, gcn_w0, gcn_w1, gcn_b0, gcn_b1, fc_w, fc_b):
    n, c_in = x.shape
    hid = gcn_w0.shape[1]
    out_ch = gcn_w1.shape[1]
    ncls = fc_w.shape[1]
    e = edge_index.shape[1]
    t = n // TM
    e_pad = ((e + 2 * CH - 1) // (2 * CH)) * (2 * CH) + 2 * CH

    src = edge_index[0].astype(jnp.int32)
    dst = edge_index[1].astype(jnp.int32)
    shift = max((n - 1).bit_length(), 1)
    key = jnp.sort((dst << shift) | src)
    dst_s = key >> shift
    src_s = key & ((1 << shift) - 1)

    cnt = jnp.zeros((n,), jnp.int32).at[dst].add(1)
    deg = 1.0 + cnt.astype(jnp.float32)
    dinv = jax.lax.rsqrt(deg)
    dinv_b = jnp.broadcast_to(dinv[:, None], (n, 128))

    node_starts = jnp.concatenate(
        [jnp.zeros((1,), jnp.int32), jnp.cumsum(cnt).astype(jnp.int32)])
    tile_starts = node_starts[::TM]
    src_pad = jnp.concatenate(
        [src_s, jnp.zeros((e_pad - e,), jnp.int32)]).reshape(-1, 2)
    src_pk = src_pad[:, 0] | (src_pad[:, 1] << 16)
    dst_pad = jnp.concatenate(
        [dst_s, jnp.full((e_pad - e,), -1, jnp.int32)])
    dst3 = dst_pad.reshape(e_pad // CH, 1, CH)

    tm0 = 1024 if n % 1024 == 0 else TM
    m0 = pl.pallas_call(
        _proj_kernel,
        out_shape=jax.ShapeDtypeStruct((n, hid), jnp.float32),
        grid=(n // tm0,),
        in_specs=[
            pl.BlockSpec((tm0, c_in), lambda i: (i, 0)),
            pl.BlockSpec((c_in, hid), lambda i: (0, 0)),
            pl.BlockSpec((tm0, 128), lambda i: (i, 0)),
        ],
        out_specs=pl.BlockSpec((tm0, hid), lambda i: (i, 0)),
        compiler_params=pltpu.CompilerParams(
            dimension_semantics=("parallel",)),
    )(x.astype(jnp.bfloat16), gcn_w0.astype(jnp.bfloat16), dinv_b)

    grid_spec1 = pltpu.PrefetchScalarGridSpec(
        num_scalar_prefetch=2,
        grid=(2, t // 2),
        in_specs=[
            pl.BlockSpec((e_pad // CH, 1, CH), lambda c, j, *_: (0, 0, 0)),
            pl.BlockSpec((n, 1, hid), lambda c, j, *_: (0, 0, 0)),
            pl.BlockSpec((TM, hid), lambda c, j, *_: (c * (t // 2) + j, 0)),
            pl.BlockSpec((TM, 128), lambda c, j, *_: (c * (t // 2) + j, 0)),
            pl.BlockSpec((1, hid), lambda c, j, *_: (0, 0)),
            pl.BlockSpec((hid, out_ch), lambda c, j, *_: (0, 0)),
        ],
        out_specs=pl.BlockSpec((TM, out_ch),
                               lambda c, j, *_: (c * (t // 2) + j, 0)),
        scratch_shapes=[pltpu.VMEM((TM, hid), jnp.float32),
                        pltpu.VMEM((2 * CH, hid), jnp.bfloat16)],
    )
    g1 = pl.pallas_call(
        _layer_kernel,
        grid_spec=grid_spec1,
        out_shape=jax.ShapeDtypeStruct((n, out_ch), jnp.float32),
        compiler_params=pltpu.CompilerParams(
            dimension_semantics=("parallel", "arbitrary")),
    )(tile_starts, src_pk, dst3, m0.reshape(n, 1, hid), m0, dinv_b,
      gcn_b0, gcn_w1.astype(jnp.bfloat16))

    grid_spec2 = pltpu.PrefetchScalarGridSpec(
        num_scalar_prefetch=2,
        grid=(2, t // 2),
        in_specs=[
            pl.BlockSpec((e_pad // CH, 1, CH), lambda c, j, *_: (0, 0, 0)),
            pl.BlockSpec((n, 1, out_ch), lambda c, j, *_: (0, 0, 0)),
            pl.BlockSpec((TM, out_ch), lambda c, j, *_: (c * (t // 2) + j, 0)),
            pl.BlockSpec((TM, 128), lambda c, j, *_: (c * (t // 2) + j, 0)),
            pl.BlockSpec((1, out_ch), lambda c, j, *_: (0, 0)),
            pl.BlockSpec((out_ch, ncls), lambda c, j, *_: (0, 0)),
            pl.BlockSpec((1, ncls), lambda c, j, *_: (0, 0)),
        ],
        out_specs=(pl.BlockSpec((TM, out_ch),
                                lambda c, j, *_: (c * (t // 2) + j, 0)),
                   pl.BlockSpec((TM, ncls),
                                lambda c, j, *_: (c * (t // 2) + j, 0))),
        scratch_shapes=[pltpu.VMEM((TM, out_ch), jnp.float32),
                        pltpu.VMEM((2 * CH, out_ch), jnp.bfloat16)],
    )
    z, probs = pl.pallas_call(
        _final_kernel,
        grid_spec=grid_spec2,
        out_shape=(jax.ShapeDtypeStruct((n, out_ch), jnp.float32),
                   jax.ShapeDtypeStruct((n, ncls), jnp.float32)),
        compiler_params=pltpu.CompilerParams(
            dimension_semantics=("parallel", "arbitrary")),
    )(tile_starts, src_pk, dst3, g1.reshape(n, 1, out_ch), g1, dinv_b,
      gcn_b1, fc_w.astype(jnp.bfloat16), fc_b)

    return z, probs


# speedup vs baseline: 1.0408x; 1.0408x over previous
import functools

import jax
import jax.numpy as jnp
from jax.experimental import pallas as pl
from jax.experimental.pallas import tpu as pltpu

TM = 128
CH = 128


def _proj_kernel(xb_ref, w_ref, dinv_ref, o_ref):
    acc = jnp.dot(xb_ref[...], w_ref[...], preferred_element_type=jnp.float32)
    o_ref[...] = acc * dinv_ref[:, :1]


def _spmm_tile(i, ts_ref, src_ref, dst_ref, m3_ref, m2_ref, acc_ref, g_ref):
    row0 = pl.multiple_of(i * TM, TM)
    acc_ref[...] = m2_ref[pl.ds(row0, TM), :]
    start = ts_ref[i]
    end = ts_ref[i + 1]
    c0 = start // CH
    c1 = (end + CH - 1) // CH
    row_iota = jax.lax.broadcasted_iota(jnp.int32, (TM, 2 * CH), 0)
    lane_iota = jax.lax.broadcasted_iota(jnp.int32, (1, 2 * CH), 1)
    base = i * TM

    def body(p, carry):
        ec = p * (2 * CH)
        for j in range(2 * CH):
            g_ref[j, :] = m3_ref[src_ref[ec + j], 0, :].astype(jnp.bfloat16)
        ev = ec + lane_iota
        dl = jnp.concatenate(
            [dst_ref[2 * p, 0, :][None, :],
             dst_ref[2 * p + 1, 0, :][None, :]], axis=1) - base
        valid = (ev >= start) & (ev < end)
        u = jnp.where(valid & (row_iota == dl), 1.0, 0.0).astype(jnp.bfloat16)
        acc_ref[...] += jnp.dot(u, g_ref[...],
                                preferred_element_type=jnp.float32)
        return carry

    jax.lax.fori_loop(c0 // 2, (c1 + 1) // 2, body, 0)
    return acc_ref[...]


def _layer_kernel(ts_ref, src_ref, dst_ref, m3_ref, m2_ref, dinv_ref, b_ref,
                  wn_ref, o_ref, acc_ref, g_ref, *, tiles_per_core):
    c = pl.program_id(0)

    def tile_body(tl, carry):
        i = c * tiles_per_core + tl
        acc = _spmm_tile(i, ts_ref, src_ref, dst_ref, m3_ref, m2_ref,
                         acc_ref, g_ref)
        row0 = pl.multiple_of(i * TM, TM)
        dinv = dinv_ref[pl.ds(row0, TM), :1]
        h = jnp.maximum(acc * dinv + b_ref[...], 0.0)
        g1 = jnp.dot(h.astype(jnp.bfloat16), wn_ref[...],
                     preferred_element_type=jnp.float32)
        o_ref[pl.ds(pl.multiple_of(tl * TM, TM), TM), :] = g1 * dinv
        return carry

    jax.lax.fori_loop(0, tiles_per_core, tile_body, 0)


def _final_kernel(ts_ref, src_ref, dst_ref, m3_ref, m2_ref, dinv_ref, b_ref,
                  wfc_ref, bfc_ref, z_ref, p_ref, acc_ref, g_ref, *,
                  tiles_per_core):
    c = pl.program_id(0)

    def tile_body(tl, carry):
        i = c * tiles_per_core + tl
        acc = _spmm_tile(i, ts_ref, src_ref, dst_ref, m3_ref, m2_ref,
                         acc_ref, g_ref)
        row0 = pl.multiple_of(i * TM, TM)
        orow = pl.multiple_of(tl * TM, TM)
        dinv = dinv_ref[pl.ds(row0, TM), :1]
        z = acc * dinv + b_ref[...]
        z_ref[pl.ds(orow, TM), :] = z
        y = jnp.dot(jnp.maximum(z, 0.0).astype(jnp.bfloat16), wfc_ref[...],
                    preferred_element_type=jnp.float32) + bfc_ref[...]
        m = jnp.max(y, axis=1, keepdims=True)
        ex = jnp.exp(y - m)
        p_ref[pl.ds(orow, TM), :] = ex / jnp.sum(ex, axis=1, keepdims=True)
        return carry

    jax.lax.fori_loop(0, tiles_per_core, tile_body, 0)


def kernel(x, edge_index, gcn_w0, gcn_w1, gcn_b0, gcn_b1, fc_w, fc_b):
    n, c_in = x.shape
    hid = gcn_w0.shape[1]
    out_ch = gcn_w1.shape[1]
    ncls = fc_w.shape[1]
    e = edge_index.shape[1]
    t = n // TM
    tpc = t // 2
    e_pad = ((e + 2 * CH - 1) // (2 * CH)) * (2 * CH) + 2 * CH

    src = edge_index[0].astype(jnp.int32)
    dst = edge_index[1].astype(jnp.int32)
    shift = max((n - 1).bit_length(), 1)
    key = jnp.sort((dst << shift) | src)
    dst_s = key >> shift
    src_s = key & ((1 << shift) - 1)

    cnt = jnp.zeros((n,), jnp.int32).at[dst].add(1)
    deg = 1.0 + cnt.astype(jnp.float32)
    dinv = jax.lax.rsqrt(deg)
    dinv_b = jnp.broadcast_to(dinv[:, None], (n, 128))

    node_starts = jnp.concatenate(
        [jnp.zeros((1,), jnp.int32), jnp.cumsum(cnt).astype(jnp.int32)])
    tile_starts = node_starts[::TM]
    src_pad = jnp.concatenate(
        [src_s, jnp.zeros((e_pad - e,), jnp.int32)])
    dst_pad = jnp.concatenate(
        [dst_s, jnp.full((e_pad - e,), -1, jnp.int32)])
    dst3 = dst_pad.reshape(e_pad // CH, 1, CH)

    tm0 = 1024 if n % 1024 == 0 else TM
    m0 = pl.pallas_call(
        _proj_kernel,
        out_shape=jax.ShapeDtypeStruct((n, hid), jnp.float32),
        grid=(n // tm0,),
        in_specs=[
            pl.BlockSpec((tm0, c_in), lambda i: (i, 0)),
            pl.BlockSpec((c_in, hid), lambda i: (0, 0)),
            pl.BlockSpec((tm0, 128), lambda i: (i, 0)),
        ],
        out_specs=pl.BlockSpec((tm0, hid), lambda i: (i, 0)),
        compiler_params=pltpu.CompilerParams(
            dimension_semantics=("parallel",)),
    )(x.astype(jnp.bfloat16), gcn_w0.astype(jnp.bfloat16), dinv_b)

    grid_spec1 = pltpu.PrefetchScalarGridSpec(
        num_scalar_prefetch=2,
        grid=(2,),
        in_specs=[
            pl.BlockSpec((e_pad // CH, 1, CH), lambda c, *_: (0, 0, 0)),
            pl.BlockSpec((n, 1, hid), lambda c, *_: (0, 0, 0)),
            pl.BlockSpec((n, hid), lambda c, *_: (0, 0)),
            pl.BlockSpec((n, 128), lambda c, *_: (0, 0)),
            pl.BlockSpec((1, hid), lambda c, *_: (0, 0)),
            pl.BlockSpec((hid, out_ch), lambda c, *_: (0, 0)),
        ],
        out_specs=pl.BlockSpec((n // 2, out_ch), lambda c, *_: (c, 0)),
        scratch_shapes=[pltpu.VMEM((TM, hid), jnp.float32),
                        pltpu.VMEM((2 * CH, hid), jnp.bfloat16)],
    )
    g1 = pl.pallas_call(
        functools.partial(_layer_kernel, tiles_per_core=tpc),
        grid_spec=grid_spec1,
        out_shape=jax.ShapeDtypeStruct((n, out_ch), jnp.float32),
        compiler_params=pltpu.CompilerParams(
            dimension_semantics=("parallel",)),
    )(tile_starts, src_pad, dst3, m0.reshape(n, 1, hid), m0, dinv_b,
      gcn_b0, gcn_w1.astype(jnp.bfloat16))

    grid_spec2 = pltpu.PrefetchScalarGridSpec(
        num_scalar_prefetch=2,
        grid=(2,),
        in_specs=[
            pl.BlockSpec((e_pad // CH, 1, CH), lambda c, *_: (0, 0, 0)),
            pl.BlockSpec((n, 1, out_ch), lambda c, *_: (0, 0, 0)),
            pl.BlockSpec((n, out_ch), lambda c, *_: (0, 0)),
            pl.BlockSpec((n, 128), lambda c, *_: (0, 0)),
            pl.BlockSpec((1, out_ch), lambda c, *_: (0, 0)),
            pl.BlockSpec((out_ch, ncls), lambda c, *_: (0, 0)),
            pl.BlockSpec((1, ncls), lambda c, *_: (0, 0)),
        ],
        out_specs=(pl.BlockSpec((n // 2, out_ch), lambda c, *_: (c, 0)),
                   pl.BlockSpec((n // 2, ncls), lambda c, *_: (c, 0))),
        scratch_shapes=[pltpu.VMEM((TM, out_ch), jnp.float32),
                        pltpu.VMEM((2 * CH, out_ch), jnp.bfloat16)],
    )
    z, probs = pl.pallas_call(
        functools.partial(_final_kernel, tiles_per_core=tpc),
        grid_spec=grid_spec2,
        out_shape=(jax.ShapeDtypeStruct((n, out_ch), jnp.float32),
                   jax.ShapeDtypeStruct((n, ncls), jnp.float32)),
        compiler_params=pltpu.CompilerParams(
            dimension_semantics=("parallel",)),
    )(tile_starts, src_pad, dst3, g1.reshape(n, 1, out_ch), g1, dinv_b,
      gcn_b1, fc_w.astype(jnp.bfloat16), fc_b)

    return z, probs


# speedup vs baseline: 1.2275x; 1.1793x over previous
import functools

import jax
import jax.numpy as jnp
from jax.experimental import pallas as pl
from jax.experimental.pallas import tpu as pltpu

TM = 128
CH = 128


def _proj_kernel(xb_ref, w_ref, dinv_ref, o_ref):
    acc = jnp.dot(xb_ref[...], w_ref[...], preferred_element_type=jnp.float32)
    o_ref[...] = acc * dinv_ref[:, :1]


def _spmm_tile(i, ts_ref, src_ref, dst_ref, m3_ref, m2_ref, acc_ref, g_ref):
    row0 = pl.multiple_of(i * TM, TM)
    acc_ref[...] = m2_ref[pl.ds(row0, TM), :]
    start = ts_ref[i]
    end = ts_ref[i + 1]
    c0 = start // CH
    c1 = (end + CH - 1) // CH
    row_iota = jax.lax.broadcasted_iota(jnp.int32, (TM, 2 * CH), 0)
    lane_iota = jax.lax.broadcasted_iota(jnp.int32, (1, 2 * CH), 1)
    base = i * TM

    def body(p, carry):
        ec = p * (2 * CH)
        for j in range(2 * CH):
            g_ref[j, :] = m3_ref[src_ref[ec + j], 0, :]
        ev = ec + lane_iota
        dl = jnp.concatenate(
            [dst_ref[2 * p, 0, :][None, :],
             dst_ref[2 * p + 1, 0, :][None, :]], axis=1) - base
        valid = (ev >= start) & (ev < end)
        u = jnp.where(valid & (row_iota == dl), 1.0, 0.0).astype(jnp.bfloat16)
        acc_ref[...] += jnp.dot(u, g_ref[...].astype(jnp.bfloat16),
                                preferred_element_type=jnp.float32)
        return carry

    jax.lax.fori_loop(c0 // 2, (c1 + 1) // 2, body, 0)
    return acc_ref[...]


def _layer_kernel(ts_ref, src_ref, dst_ref, m3_ref, m2_ref, dinv_ref, b_ref,
                  wn_ref, o_ref, acc_ref, g_ref, *, tiles_per_core):
    c = pl.program_id(0)

    def tile_body(tl, carry):
        i = c * tiles_per_core + tl
        acc = _spmm_tile(i, ts_ref, src_ref, dst_ref, m3_ref, m2_ref,
                         acc_ref, g_ref)
        row0 = pl.multiple_of(i * TM, TM)
        dinv = dinv_ref[pl.ds(row0, TM), :1]
        h = jnp.maximum(acc * dinv + b_ref[...], 0.0)
        g1 = jnp.dot(h.astype(jnp.bfloat16), wn_ref[...],
                     preferred_element_type=jnp.float32)
        o_ref[pl.ds(pl.multiple_of(tl * TM, TM), TM), :] = g1 * dinv
        return carry

    jax.lax.fori_loop(0, tiles_per_core, tile_body, 0)


def _final_kernel(ts_ref, src_ref, dst_ref, m3_ref, m2_ref, dinv_ref, b_ref,
                  wfc_ref, bfc_ref, z_ref, p_ref, acc_ref, g_ref, *,
                  tiles_per_core):
    c = pl.program_id(0)

    def tile_body(tl, carry):
        i = c * tiles_per_core + tl
        acc = _spmm_tile(i, ts_ref, src_ref, dst_ref, m3_ref, m2_ref,
                         acc_ref, g_ref)
        row0 = pl.multiple_of(i * TM, TM)
        orow = pl.multiple_of(tl * TM, TM)
        dinv = dinv_ref[pl.ds(row0, TM), :1]
        z = acc * dinv + b_ref[...]
        z_ref[pl.ds(orow, TM), :] = z
        y = jnp.dot(jnp.maximum(z, 0.0).astype(jnp.bfloat16), wfc_ref[...],
                    preferred_element_type=jnp.float32) + bfc_ref[...]
        m = jnp.max(y, axis=1, keepdims=True)
        ex = jnp.exp(y - m)
        p_ref[pl.ds(orow, TM), :] = ex / jnp.sum(ex, axis=1, keepdims=True)
        return carry

    jax.lax.fori_loop(0, tiles_per_core, tile_body, 0)


def kernel(x, edge_index, gcn_w0, gcn_w1, gcn_b0, gcn_b1, fc_w, fc_b):
    n, c_in = x.shape
    hid = gcn_w0.shape[1]
    out_ch = gcn_w1.shape[1]
    ncls = fc_w.shape[1]
    e = edge_index.shape[1]
    t = n // TM
    tpc = t // 2
    e_pad = ((e + 2 * CH - 1) // (2 * CH)) * (2 * CH) + 2 * CH

    src = edge_index[0].astype(jnp.int32)
    dst = edge_index[1].astype(jnp.int32)
    shift = max((n - 1).bit_length(), 1)
    key = jnp.sort((dst << shift) | src)
    dst_s = key >> shift
    src_s = key & ((1 << shift) - 1)

    cnt = jnp.zeros((n,), jnp.int32).at[dst].add(1)
    deg = 1.0 + cnt.astype(jnp.float32)
    dinv = jax.lax.rsqrt(deg)
    dinv_b = jnp.broadcast_to(dinv[:, None], (n, 128))

    node_starts = jnp.concatenate(
        [jnp.zeros((1,), jnp.int32), jnp.cumsum(cnt).astype(jnp.int32)])
    tile_starts = node_starts[::TM]
    src_pad = jnp.concatenate(
        [src_s, jnp.zeros((e_pad - e,), jnp.int32)])
    dst_pad = jnp.concatenate(
        [dst_s, jnp.full((e_pad - e,), -1, jnp.int32)])
    dst3 = dst_pad.reshape(e_pad // CH, 1, CH)

    tm0 = 1024 if n % 1024 == 0 else TM
    m0 = pl.pallas_call(
        _proj_kernel,
        out_shape=jax.ShapeDtypeStruct((n, hid), jnp.float32),
        grid=(n // tm0,),
        in_specs=[
            pl.BlockSpec((tm0, c_in), lambda i: (i, 0)),
            pl.BlockSpec((c_in, hid), lambda i: (0, 0)),
            pl.BlockSpec((tm0, 128), lambda i: (i, 0)),
        ],
        out_specs=pl.BlockSpec((tm0, hid), lambda i: (i, 0)),
        compiler_params=pltpu.CompilerParams(
            dimension_semantics=("parallel",)),
    )(x.astype(jnp.bfloat16), gcn_w0.astype(jnp.bfloat16), dinv_b)

    grid_spec1 = pltpu.PrefetchScalarGridSpec(
        num_scalar_prefetch=2,
        grid=(2,),
        in_specs=[
            pl.BlockSpec((e_pad // CH, 1, CH), lambda c, *_: (0, 0, 0)),
            pl.BlockSpec((n, 1, hid), lambda c, *_: (0, 0, 0)),
            pl.BlockSpec((n, hid), lambda c, *_: (0, 0)),
            pl.BlockSpec((n, 128), lambda c, *_: (0, 0)),
            pl.BlockSpec((1, hid), lambda c, *_: (0, 0)),
            pl.BlockSpec((hid, out_ch), lambda c, *_: (0, 0)),
        ],
        out_specs=pl.BlockSpec((n // 2, out_ch), lambda c, *_: (c, 0)),
        scratch_shapes=[pltpu.VMEM((TM, hid), jnp.float32),
                        pltpu.VMEM((2 * CH, hid), jnp.float32)],
    )
    g1 = pl.pallas_call(
        functools.partial(_layer_kernel, tiles_per_core=tpc),
        grid_spec=grid_spec1,
        out_shape=jax.ShapeDtypeStruct((n, out_ch), jnp.float32),
        compiler_params=pltpu.CompilerParams(
            dimension_semantics=("parallel",)),
    )(tile_starts, src_pad, dst3, m0.reshape(n, 1, hid), m0, dinv_b,
      gcn_b0, gcn_w1.astype(jnp.bfloat16))

    grid_spec2 = pltpu.PrefetchScalarGridSpec(
        num_scalar_prefetch=2,
        grid=(2,),
        in_specs=[
            pl.BlockSpec((e_pad // CH, 1, CH), lambda c, *_: (0, 0, 0)),
            pl.BlockSpec((n, 1, out_ch), lambda c, *_: (0, 0, 0)),
            pl.BlockSpec((n, out_ch), lambda c, *_: (0, 0)),
            pl.BlockSpec((n, 128), lambda c, *_: (0, 0)),
            pl.BlockSpec((1, out_ch), lambda c, *_: (0, 0)),
            pl.BlockSpec((out_ch, ncls), lambda c, *_: (0, 0)),
            pl.BlockSpec((1, ncls), lambda c, *_: (0, 0)),
        ],
        out_specs=(pl.BlockSpec((n // 2, out_ch), lambda c, *_: (c, 0)),
                   pl.BlockSpec((n // 2, ncls), lambda c, *_: (c, 0))),
        scratch_shapes=[pltpu.VMEM((TM, out_ch), jnp.float32),
                        pltpu.VMEM((2 * CH, out_ch), jnp.float32)],
    )
    z, probs = pl.pallas_call(
        functools.partial(_final_kernel, tiles_per_core=tpc),
        grid_spec=grid_spec2,
        out_shape=(jax.ShapeDtypeStruct((n, out_ch), jnp.float32),
                   jax.ShapeDtypeStruct((n, ncls), jnp.float32)),
        compiler_params=pltpu.CompilerParams(
            dimension_semantics=("parallel",)),
    )(tile_starts, src_pad, dst3, g1.reshape(n, 1, out_ch), g1, dinv_b,
      gcn_b1, fc_w.astype(jnp.bfloat16), fc_b)

    return z, probs
